# baseline (speedup 1.0000x reference)
import functools

import jax
import jax.numpy as jnp
from jax import lax
from jax.experimental import pallas as pl
from jax.experimental.pallas import tpu as pltpu
from jax.experimental.pallas import tpu_sc as plsc

N = 10000
D_IN = 256
HID = 128
HEADS = 4
NCLS = 64
NEG = 0.2

NP = 10240
ETOT = 170000
CK = 48
EP1 = 10752
CKB = 96
C2 = 56
EP2 = C2 * CKB
KTH = 81921
I32MIN = jnp.iinfo(jnp.int32).min
I32MAX = jnp.iinfo(jnp.int32).max


def _thr_body(s_ref, o_ref):
    bits = lax.bitcast_convert_type(s_ref[...], jnp.int32)
    keys = jnp.where(bits >= 0, bits, I32MIN - bits)
    cnt_neg = jnp.sum((keys < 0).astype(jnp.int32))
    in_neg = cnt_neg >= KTH
    lo0 = jnp.where(in_neg, I32MIN, 0).astype(jnp.int32)
    hi0 = jnp.where(in_neg, -1, I32MAX).astype(jnp.int32)

    def step(_, carry):
        lo, hi = carry
        mid = lo + lax.div(hi - lo, jnp.int32(2))
        cnt = jnp.sum((keys <= mid).astype(jnp.int32))
        ok = cnt >= KTH
        return (jnp.where(ok, lo, mid + 1), jnp.where(ok, mid, hi))

    lo, hi = lax.fori_loop(0, 31, step, (lo0, hi0))
    v = lo
    tbits = jnp.where(v >= 0, v, I32MIN - v).reshape(1, 1)
    o_ref[...] = lax.bitcast_convert_type(tbits, jnp.float32)


def _thr_call(scores_2d):
    return pl.pallas_call(
        _thr_body,
        out_shape=jax.ShapeDtypeStruct((1, 1), jnp.float32),
        in_specs=[pl.BlockSpec(memory_space=pltpu.VMEM)],
        out_specs=pl.BlockSpec(memory_space=pltpu.VMEM),
    )(scores_2d)


def _l1_body(thr_ref, x_ref, w_ref, s_ref, al_ref, ar_ref,
             z_ref, el_ref, er_ref, elm_ref, erm_ref):
    thr = thr_ref[0, 0]
    w = w_ref[...] * (s_ref[...] > thr).astype(jnp.float32)
    zb = jnp.dot(x_ref[...], w, preferred_element_type=jnp.float32)

    els, ers = [], []
    for h in range(HEADS):
        zh = zb[:, HID * h:HID * (h + 1)]
        els.append(jnp.sum(zh * al_ref[h:h + 1, :], axis=1, keepdims=True))
        ers.append(jnp.sum(zh * ar_ref[h:h + 1, :], axis=1, keepdims=True))
    el = jnp.concatenate(els, axis=1)
    er = jnp.concatenate(ers, axis=1)

    i = pl.program_id(0)
    ridx = i * 128 + lax.broadcasted_iota(jnp.int32, (128, HEADS), 0)
    valid = ridx < N
    el = jnp.where(valid, el, -1e30)
    er = jnp.where(valid, er, -1e30)

    for h in range(HEADS):
        z_ref[h] = zb[:, HID * h:HID * (h + 1)]
    el_ref[...] = el
    er_ref[...] = er
    ml = jnp.max(el, axis=0, keepdims=True)
    mr = jnp.max(er, axis=0, keepdims=True)

    @pl.when(i == 0)
    def _():
        elm_ref[...] = ml
        erm_ref[...] = mr

    @pl.when(i > 0)
    def _():
        elm_ref[...] = jnp.maximum(elm_ref[...], ml)
        erm_ref[...] = jnp.maximum(erm_ref[...], mr)


def _l1_call(thr, hp, W0, score0, attn_l0, attn_r0):
    grid = (NP // 128,)
    return pl.pallas_call(
        _l1_body,
        grid=grid,
        in_specs=[
            pl.BlockSpec(memory_space=pltpu.SMEM),
            pl.BlockSpec((128, D_IN), lambda i: (i, 0)),
            pl.BlockSpec((D_IN, HEADS * HID), lambda i: (0, 0)),
            pl.BlockSpec((D_IN, HEADS * HID), lambda i: (0, 0)),
            pl.BlockSpec((HEADS, HID), lambda i: (0, 0)),
            pl.BlockSpec((HEADS, HID), lambda i: (0, 0)),
        ],
        out_specs=[
            pl.BlockSpec((HEADS, 128, HID), lambda i: (0, i, 0)),
            pl.BlockSpec((128, HEADS), lambda i: (i, 0)),
            pl.BlockSpec((128, HEADS), lambda i: (i, 0)),
            pl.BlockSpec((1, HEADS), lambda i: (0, 0)),
            pl.BlockSpec((1, HEADS), lambda i: (0, 0)),
        ],
        out_shape=[
            jax.ShapeDtypeStruct((HEADS, NP, HID), jnp.float32),
            jax.ShapeDtypeStruct((NP, HEADS), jnp.float32),
            jax.ShapeDtypeStruct((NP, HEADS), jnp.float32),
            jax.ShapeDtypeStruct((1, HEADS), jnp.float32),
            jax.ShapeDtypeStruct((1, HEADS), jnp.float32),
        ],
    )(thr, hp, W0, score0, attn_l0, attn_r0)


SCN = 1024
NSC = (16 * EP1) // SCN
CAP = 11776
CAPP = CAP + 128
MSK = 16383


def _sc1_body(z_hbm, el_hbm, er_hbm, m_hbm, wpk_hbm,
              acc_out, den_out,
              elv, erloc, mspv, binv, idxo0, idxo1, dlv0, dlv1, av0, av1,
              rows0, rows1, sb0, sb1, accf, den_l,
              gs0, gs1, sb0s, sb1s):
    c = lax.axis_index("c")
    s = lax.axis_index("s")
    lo = s * 640
    idxob = (idxo0, idxo1)
    dlvb = (dlv0, dlv1)
    avb = (av0, av1)
    rowsb = (rows0, rows1)
    sbb = (sb0, sb1)
    gsem = (gs0, gs1)
    ssb = (sb0s, sb1s)

    def scan_issue(k, b):
        pltpu.async_copy(wpk_hbm.at[pl.ds(k * SCN, SCN)], sbb[b], ssb[b])

    def scan_proc(b, cnt):
        pltpu.make_async_copy(wpk_hbm.at[pl.ds(0, SCN)], sbb[b],
                              ssb[b]).wait()

        def grp(g, cnt):
            w = sbb[b][pl.ds(16 * g, 16)]
            d = lax.bitwise_and(w, MSK)
            m = jnp.logical_and(d >= lo, d < lo + 640)
            plsc.store_compressed(binv.at[pl.ds(cnt, 16)], w, mask=m)
            return cnt + jnp.max(plsc.all_reduce_population_count(m))

        return pl.loop(0, SCN // 16, init_carry=cnt)(grp)

    scan_issue(0, 0)
    scan_issue(1, 1)

    @pl.loop(0, NSC // 2 - 1, init_carry=jnp.int32(0))
    def cnt(jj, cnt):
        cnt = scan_proc(0, cnt)
        scan_issue(2 * jj + 2, 0)
        cnt = scan_proc(1, cnt)
        scan_issue(2 * jj + 3, 1)
        return cnt

    cnt = scan_proc(0, cnt)
    cnt = scan_proc(1, cnt)

    dum = jnp.int32(N * (MSK + 1) + 0) + lo
    for g in range(7):
        binv[pl.ds(cnt + 16 * g, 16)] = jnp.zeros((16,), jnp.int32) + dum
    nch = lax.div(cnt + 2 * CK - 1, jnp.int32(2 * CK)) * 2

    lane16 = lax.iota(jnp.int32, 16)
    m0 = lane16 == 0
    for j in range(2):
        h = 2 * c + j
        hoff = h * NP

        @pl.loop(0, 640)
        def _(r):
            for g in range(HID // 16):
                accf[pl.ds(r * HID + 16 * g, 16)] = jnp.zeros((16,),
                                                              jnp.float32)

        @pl.loop(0, 40)
        def _(g):
            den_l[pl.ds(16 * g, 16)] = jnp.zeros((16,), jnp.float32)

        pltpu.sync_copy(el_hbm.at[h], elv)
        pltpu.sync_copy(er_hbm.at[h].at[pl.ds(lo, 640)], erloc)
        pltpu.sync_copy(m_hbm.at[h], mspv)
        msp = mspv[...]

        def gpart(k, b):
            for g in range(CK // 16):
                sl = pl.ds(16 * g, 16)
                w = binv[pl.ds(k * CK + 16 * g, 16)]
                srcg = lax.shift_right_logical(w, 14)
                dlg = lax.bitwise_and(w, MSK) - lo
                idxob[b][sl] = srcg + hoff
                dlvb[b][sl] = dlg
                e = (plsc.load_gather(elv, [srcg])
                     + plsc.load_gather(erloc, [dlg]))
                e = jnp.where(e > 0, e, NEG * e) - msp
                avb[b][sl] = jnp.exp(e)
            pltpu.async_copy(z_hbm.at[idxob[b]], rowsb[b], gsem[b])

        def spart(k, b):
            pltpu.make_async_copy(z_hbm.at[idxob[b]], rowsb[b],
                                  gsem[b]).wait()
            r = rowsb[b]
            a = avb[b]
            dv = dlvb[b]

            @pl.loop(0, CK)
            def _(i):
                sp = jnp.full((16,), 0, jnp.int32) + i
                ai = plsc.load_gather(a, [sp])
                dl = plsc.load_gather(dv, [sp])
                base = dl * HID
                for g in range(HID // 16):
                    idx = base + (lane16 + 16 * g)
                    plsc.addupdate_scatter(accf, [idx],
                                           r[i, pl.ds(16 * g, 16)] * ai)
                plsc.addupdate_scatter(den_l, [dl], ai, mask=m0)

        gpart(0, 0)
        gpart(1, 1)
        spart(0, 0)

        @pl.loop(0, lax.div(nch - 2, jnp.int32(2)))
        def _(jj):
            k0 = 2 * jj + 2
            gpart(k0, 0)
            spart(k0 - 1, 1)
            gpart(k0 + 1, 1)
            spart(k0, 0)

        spart(nch - 1, 1)

        pltpu.sync_copy(accf, acc_out.at[h].at[pl.ds(lo * HID, 640 * HID)])
        pltpu.sync_copy(den_l, den_out.at[h].at[pl.ds(lo, 640)])


@functools.cache
def _sc1_kernel():
    return functools.partial(
        pl.kernel,
        out_type=(jax.ShapeDtypeStruct((HEADS, NP * HID), jnp.float32),
                  jax.ShapeDtypeStruct((HEADS, NP), jnp.float32)),
        mesh=plsc.VectorSubcoreMesh(core_axis_name="c", subcore_axis_name="s"),
        compiler_params=pltpu.CompilerParams(needs_layout_passes=False,
                                             use_tc_tiling_on_sc=False),
        scratch_types=[
            pltpu.VMEM((NP,), jnp.float32),
            pltpu.VMEM((640,), jnp.float32),
            pltpu.VMEM((16,), jnp.float32),
            pltpu.VMEM((CAPP,), jnp.int32),
            pltpu.VMEM((CK,), jnp.int32),
            pltpu.VMEM((CK,), jnp.int32),
            pltpu.VMEM((CK,), jnp.int32),
            pltpu.VMEM((CK,), jnp.int32),
            pltpu.VMEM((CK,), jnp.float32),
            pltpu.VMEM((CK,), jnp.float32),
            pltpu.VMEM((CK, HID), jnp.float32),
            pltpu.VMEM((CK, HID), jnp.float32),
            pltpu.VMEM((SCN,), jnp.int32),
            pltpu.VMEM((SCN,), jnp.int32),
            pltpu.VMEM((640 * HID,), jnp.float32),
            pltpu.VMEM((640,), jnp.float32),
            pltpu.SemaphoreType.DMA,
            pltpu.SemaphoreType.DMA,
            pltpu.SemaphoreType.DMA,
            pltpu.SemaphoreType.DMA,
        ],
    )(_sc1_body)


def _sc1_call(*args):
    return _sc1_kernel()(*args)


def _l2_body(thr_ref, acc_ref, den_ref, w_ref, s_ref, al_ref, ar_ref,
             z1_ref, el1_ref, er1_ref, m1l_ref, m1r_ref):
    parts = []
    for h in range(HEADS):
        d = den_ref[:, h:h + 1]
        x = acc_ref[h] / jnp.maximum(d, 1e-9)
        parts.append(jnp.where(x > 0, x, jnp.exp(jnp.minimum(x, 0.0)) - 1.0))
    h1 = jnp.concatenate(parts, axis=1)
    thr = thr_ref[0, 0]
    w = w_ref[...] * (s_ref[...] > thr).astype(jnp.float32)
    z1 = jnp.dot(h1, w, preferred_element_type=jnp.float32)
    el1 = jnp.sum(z1 * al_ref[...], axis=1, keepdims=True)
    er1 = jnp.sum(z1 * ar_ref[...], axis=1, keepdims=True)
    i = pl.program_id(0)
    ridx = i * 128 + lax.broadcasted_iota(jnp.int32, (128, 1), 0)
    valid = ridx < N
    el1 = jnp.where(valid, el1, -1e30)
    er1 = jnp.where(valid, er1, -1e30)
    z1_ref[...] = z1
    el1_ref[...] = el1
    er1_ref[...] = er1
    ml = jnp.max(el1)
    mr = jnp.max(er1)

    @pl.when(i == 0)
    def _():
        m1l_ref[0, 0] = ml
        m1r_ref[0, 0] = mr

    @pl.when(i > 0)
    def _():
        m1l_ref[0, 0] = jnp.maximum(m1l_ref[0, 0], ml)
        m1r_ref[0, 0] = jnp.maximum(m1r_ref[0, 0], mr)


def _l2_call(thr, acc0, den0T, W1, score1, attn_l1, attn_r1):
    grid = (NP // 128,)
    return pl.pallas_call(
        _l2_body,
        grid=grid,
        in_specs=[
            pl.BlockSpec(memory_space=pltpu.SMEM),
            pl.BlockSpec((HEADS, 128, HID), lambda i: (0, i, 0)),
            pl.BlockSpec((128, HEADS), lambda i: (i, 0)),
            pl.BlockSpec((HEADS * HID, NCLS), lambda i: (0, 0)),
            pl.BlockSpec((HEADS * HID, NCLS), lambda i: (0, 0)),
            pl.BlockSpec((1, NCLS), lambda i: (0, 0)),
            pl.BlockSpec((1, NCLS), lambda i: (0, 0)),
        ],
        out_specs=[
            pl.BlockSpec((128, NCLS), lambda i: (i, 0)),
            pl.BlockSpec((128, 1), lambda i: (i, 0)),
            pl.BlockSpec((128, 1), lambda i: (i, 0)),
            pl.BlockSpec(memory_space=pltpu.SMEM),
            pl.BlockSpec(memory_space=pltpu.SMEM),
        ],
        out_shape=[
            jax.ShapeDtypeStruct((NP, NCLS), jnp.float32),
            jax.ShapeDtypeStruct((NP, 1), jnp.float32),
            jax.ShapeDtypeStruct((NP, 1), jnp.float32),
            jax.ShapeDtypeStruct((1, 1), jnp.float32),
            jax.ShapeDtypeStruct((1, 1), jnp.float32),
        ],
    )(thr, acc0, den0T, W1, score1, attn_l1, attn_r1)


def _sc2_body(z_hbm, el_hbm, er_hbm, m_hbm, src_hbm, dst_hbm,
              acc_out, den_out,
              srcv, dstv, elv, erv, mspv, av0, av1, rows0, rows1, zden,
              acc_sh, den_sh, gs0, gs1, ss0, ss1, ds0, ds1):
    c = lax.axis_index("c")
    s = lax.axis_index("s")
    w = c * 16 + s
    rowsb = (rows0, rows1)
    avb = (av0, av1)
    gsem = (gs0, gs1)
    ssem = (ss0, ss1)
    dsem = (ds0, ds1)

    pltpu.sync_copy(src_hbm.at[w], srcv)
    pltpu.sync_copy(dst_hbm.at[w], dstv)

    def zero_rows(r):
        @pl.loop(0, CKB)
        def _(i):
            for g in range(NCLS // 16):
                r[i, pl.ds(16 * g, 16)] = jnp.zeros((16,), jnp.float32)

    zero_rows(rows0)
    zero_rows(rows1)

    @pl.loop(0, 40)
    def _(g):
        zden[pl.ds(16 * g, 16)] = jnp.zeros((16,), jnp.float32)

    @pl.loop(0, 6)
    def _(b):
        pltpu.sync_copy(rows0, acc_sh.at[pl.ds(s * 640 + b * CKB, CKB)])

    pltpu.sync_copy(rows0.at[pl.ds(0, 64)],
                    acc_sh.at[pl.ds(s * 640 + 6 * CKB, 64)])
    pltpu.sync_copy(zden, den_sh.at[pl.ds(s * 640, 640)])
    pltpu.sync_copy(el_hbm, elv)
    pltpu.sync_copy(er_hbm, erv)
    pltpu.sync_copy(m_hbm, mspv)
    plsc.subcore_barrier()
    msp = mspv[...]

    def gpart(k, b):
        pltpu.make_async_copy(rowsb[b], acc_sh.at[dstv.at[0]],
                              ssem[b]).wait()
        pltpu.make_async_copy(avb[b], den_sh.at[dstv.at[0]],
                              dsem[b]).wait()
        pltpu.async_copy(z_hbm.at[srcv.at[k]], rowsb[b], gsem[b])
        for g in range(CKB // 16):
            sl = pl.ds(16 * g, 16)
            e = (plsc.load_gather(elv, [srcv[k, sl]])
                 + plsc.load_gather(erv, [dstv[k, sl]]))
            e = jnp.where(e > 0, e, NEG * e) - msp
            avb[b][sl] = jnp.exp(e)

    def spart(k, b):
        pltpu.make_async_copy(z_hbm.at[srcv.at[k]], rowsb[b],
                              gsem[b]).wait()
        r = rowsb[b]
        a = avb[b]

        @pl.loop(0, CKB)
        def _(i):
            ai = plsc.load_gather(a, [jnp.full((16,), 0, jnp.int32) + i])
            for g in range(NCLS // 16):
                sl = pl.ds(16 * g, 16)
                r[i, sl] = r[i, sl] * ai

        pltpu.async_copy(r, acc_sh.at[dstv.at[k]], ssem[b], add=True)
        pltpu.async_copy(a, den_sh.at[dstv.at[k]], dsem[b], add=True)

    pltpu.async_copy(rows0, acc_sh.at[dstv.at[0]], ss0, add=True)
    pltpu.async_copy(rows1, acc_sh.at[dstv.at[0]], ss1, add=True)
    pltpu.async_copy(zden.at[pl.ds(0, CKB)], den_sh.at[dstv.at[0]],
                     ds0, add=True)
    pltpu.async_copy(zden.at[pl.ds(0, CKB)], den_sh.at[dstv.at[0]],
                     ds1, add=True)

    gpart(0, 0)
    gpart(1, 1)
    spart(0, 0)

    @pl.loop(0, (C2 - 2) // 2)
    def _(jj):
        k0 = 2 * jj + 2
        gpart(k0, 0)
        spart(k0 - 1, 1)
        gpart(k0 + 1, 1)
        spart(k0, 0)

    spart(C2 - 1, 1)
    for b in range(2):
        pltpu.make_async_copy(rowsb[b], acc_sh.at[dstv.at[0]],
                              ssem[b]).wait()
        pltpu.make_async_copy(avb[b], den_sh.at[dstv.at[0]],
                              dsem[b]).wait()
    plsc.subcore_barrier()

    @pl.loop(0, 6)
    def _(b):
        sl = pl.ds(s * 640 + b * CKB, CKB)
        pltpu.sync_copy(acc_sh.at[sl], acc_out.at[c].at[sl])

    sl64 = pl.ds(s * 640 + 6 * CKB, 64)
    pltpu.sync_copy(acc_sh.at[sl64], acc_out.at[c].at[sl64])
    pltpu.sync_copy(den_sh.at[pl.ds(s * 640, 640)],
                    den_out.at[c].at[pl.ds(s * 640, 640)])


@functools.cache
def _sc2_kernel():
    return functools.partial(
        pl.kernel,
        out_type=(jax.ShapeDtypeStruct((2, NP, NCLS), jnp.float32),
                  jax.ShapeDtypeStruct((2, NP), jnp.float32)),
        mesh=plsc.VectorSubcoreMesh(core_axis_name="c", subcore_axis_name="s"),
        compiler_params=pltpu.CompilerParams(needs_layout_passes=False,
                                             use_tc_tiling_on_sc=False),
        scratch_types=[
            pltpu.VMEM((C2, CKB), jnp.int32),
            pltpu.VMEM((C2, CKB), jnp.int32),
            pltpu.VMEM((NP,), jnp.float32),
            pltpu.VMEM((NP,), jnp.float32),
            pltpu.VMEM((16,), jnp.float32),
            pltpu.VMEM((CKB,), jnp.float32),
            pltpu.VMEM((CKB,), jnp.float32),
            pltpu.VMEM((CKB, NCLS), jnp.float32),
            pltpu.VMEM((CKB, NCLS), jnp.float32),
            pltpu.VMEM((640,), jnp.float32),
            pltpu.VMEM_SHARED((NP, NCLS), jnp.float32),
            pltpu.VMEM_SHARED((NP,), jnp.float32),
            pltpu.SemaphoreType.DMA,
            pltpu.SemaphoreType.DMA,
            pltpu.SemaphoreType.DMA,
            pltpu.SemaphoreType.DMA,
            pltpu.SemaphoreType.DMA,
            pltpu.SemaphoreType.DMA,
        ],
    )(_sc2_body)


def _sc2_call(*args):
    return _sc2_kernel()(*args)


def _fin_body(acc_ref, den_ref, o_ref):
    num = acc_ref[0] + acc_ref[1]
    den = den_ref[:, 0:1] + den_ref[:, 1:2]
    o_ref[...] = num / jnp.maximum(den, 1e-9)


def _fin_call(acc1, den1T):
    grid = (NP // 128,)
    return pl.pallas_call(
        _fin_body,
        grid=grid,
        in_specs=[
            pl.BlockSpec((2, 128, NCLS), lambda i: (0, i, 0)),
            pl.BlockSpec((128, 2), lambda i: (i, 0)),
        ],
        out_specs=pl.BlockSpec((128, NCLS), lambda i: (i, 0)),
        out_shape=jax.ShapeDtypeStruct((NP, NCLS), jnp.float32),
    )(acc1, den1T)


def kernel(h, edge_index, W0, score0, attn_l0, attn_r0, W1, score1,
           attn_l1, attn_r1):
    f32 = jnp.float32
    i32 = jnp.int32

    scores = jnp.concatenate(
        [score0.reshape(-1), score1.reshape(-1)]).reshape(1280, 128)
    thr = _thr_call(scores)

    hp = jnp.pad(h, ((0, NP - N), (0, 0)))
    z0, el0, er0, elm, erm = _l1_call(thr, hp, W0, score0, attn_l0, attn_r0)
    msp0 = jnp.maximum(elm + erm, 0.0).reshape(HEADS, 1) * jnp.ones((1, 16), f32)

    loops = jnp.arange(N, dtype=i32)
    src = jnp.concatenate([edge_index[0], loops])
    dst = jnp.concatenate([edge_index[1], loops])
    pad1 = 16 * EP1 - ETOT
    padv = jnp.full((pad1,), N, i32)
    srcp = jnp.concatenate([src, padv])
    dstp = jnp.concatenate([dst, padv])
    wpk = srcp * jnp.int32(16384) + dstp

    zflat = z0.reshape(HEADS * NP, HID)
    accw, den0 = _sc1_call(zflat, el0.T, er0.T, msp0, wpk)
    acc0 = accw.reshape(HEADS, NP, HID)

    z1, el1, er1, m1l, m1r = _l2_call(thr, acc0, den0.T, W1, score1,
                                      attn_l1, attn_r1)
    msp1 = jnp.maximum(m1l[0, 0] + m1r[0, 0], 0.0) * jnp.ones((16,), f32)

    src2 = jnp.concatenate([src, padv]).reshape(32, C2, CKB)
    dst2 = jnp.concatenate([dst, padv]).reshape(32, C2, CKB)
    acc1, den1 = _sc2_call(z1, el1.reshape(NP), er1.reshape(NP), msp1,
                           src2, dst2)

    out = _fin_call(acc1, den1.T)
    return out[:N]


# speedup vs baseline: 1.4924x; 1.4924x over previous
import functools

import jax
import jax.numpy as jnp
from jax import lax
from jax.experimental import pallas as pl
from jax.experimental.pallas import tpu as pltpu
from jax.experimental.pallas import tpu_sc as plsc

N = 10000
D_IN = 256
HID = 128
HEADS = 4
NCLS = 64
NEG = 0.2

NP = 10240
ETOT = 170000
CK = 96
C1 = 112
EP1 = C1 * CK
C2 = 56
EP2 = C2 * CK
KTH = 81921
I32MIN = jnp.iinfo(jnp.int32).min
I32MAX = jnp.iinfo(jnp.int32).max


def _thr_body(s_ref, o_ref):
    bits = lax.bitcast_convert_type(s_ref[...], jnp.int32)
    keys = jnp.where(bits >= 0, bits, I32MIN - bits)
    cnt_neg = jnp.sum((keys < 0).astype(jnp.int32))
    in_neg = cnt_neg >= KTH
    lo0 = jnp.where(in_neg, I32MIN, 0).astype(jnp.int32)
    hi0 = jnp.where(in_neg, -1, I32MAX).astype(jnp.int32)

    def step(_, carry):
        lo, hi = carry
        mid = lo + lax.div(hi - lo, jnp.int32(2))
        cnt = jnp.sum((keys <= mid).astype(jnp.int32))
        ok = cnt >= KTH
        return (jnp.where(ok, lo, mid + 1), jnp.where(ok, mid, hi))

    lo, hi = lax.fori_loop(0, 31, step, (lo0, hi0))
    v = lo
    tbits = jnp.where(v >= 0, v, I32MIN - v).reshape(1, 1)
    o_ref[...] = lax.bitcast_convert_type(tbits, jnp.float32)


def _thr_call(scores_2d):
    return pl.pallas_call(
        _thr_body,
        out_shape=jax.ShapeDtypeStruct((1, 1), jnp.float32),
        in_specs=[pl.BlockSpec(memory_space=pltpu.VMEM)],
        out_specs=pl.BlockSpec(memory_space=pltpu.VMEM),
    )(scores_2d)


def _l1_body(thr_ref, x_ref, w_ref, s_ref, al_ref, ar_ref,
             z_ref, el_ref, er_ref, elm_ref, erm_ref):
    thr = thr_ref[0, 0]
    w = w_ref[...] * (s_ref[...] > thr).astype(jnp.float32)
    zb = jnp.dot(x_ref[...], w, preferred_element_type=jnp.float32)

    els, ers = [], []
    for h in range(HEADS):
        zh = zb[:, HID * h:HID * (h + 1)]
        els.append(jnp.sum(zh * al_ref[h:h + 1, :], axis=1, keepdims=True))
        ers.append(jnp.sum(zh * ar_ref[h:h + 1, :], axis=1, keepdims=True))
    el = jnp.concatenate(els, axis=1)
    er = jnp.concatenate(ers, axis=1)

    i = pl.program_id(0)
    ridx = i * 128 + lax.broadcasted_iota(jnp.int32, (128, HEADS), 0)
    valid = ridx < N
    el = jnp.where(valid, el, -1e30)
    er = jnp.where(valid, er, -1e30)

    for h in range(HEADS):
        z_ref[h] = zb[:, HID * h:HID * (h + 1)]
    el_ref[...] = el
    er_ref[...] = er
    ml = jnp.max(el, axis=0, keepdims=True)
    mr = jnp.max(er, axis=0, keepdims=True)

    @pl.when(i == 0)
    def _():
        elm_ref[...] = ml
        erm_ref[...] = mr

    @pl.when(i > 0)
    def _():
        elm_ref[...] = jnp.maximum(elm_ref[...], ml)
        erm_ref[...] = jnp.maximum(erm_ref[...], mr)


def _l1_call(thr, hp, W0, score0, attn_l0, attn_r0):
    grid = (NP // 128,)
    return pl.pallas_call(
        _l1_body,
        grid=grid,
        in_specs=[
            pl.BlockSpec(memory_space=pltpu.SMEM),
            pl.BlockSpec((128, D_IN), lambda i: (i, 0)),
            pl.BlockSpec((D_IN, HEADS * HID), lambda i: (0, 0)),
            pl.BlockSpec((D_IN, HEADS * HID), lambda i: (0, 0)),
            pl.BlockSpec((HEADS, HID), lambda i: (0, 0)),
            pl.BlockSpec((HEADS, HID), lambda i: (0, 0)),
        ],
        out_specs=[
            pl.BlockSpec((HEADS, 128, HID), lambda i: (0, i, 0)),
            pl.BlockSpec((128, HEADS), lambda i: (i, 0)),
            pl.BlockSpec((128, HEADS), lambda i: (i, 0)),
            pl.BlockSpec((1, HEADS), lambda i: (0, 0)),
            pl.BlockSpec((1, HEADS), lambda i: (0, 0)),
        ],
        out_shape=[
            jax.ShapeDtypeStruct((HEADS, NP, HID), jnp.float32),
            jax.ShapeDtypeStruct((NP, HEADS), jnp.float32),
            jax.ShapeDtypeStruct((NP, HEADS), jnp.float32),
            jax.ShapeDtypeStruct((1, HEADS), jnp.float32),
            jax.ShapeDtypeStruct((1, HEADS), jnp.float32),
        ],
    )(thr, hp, W0, score0, attn_l0, attn_r0)


def _sc0_body(el_hbm, er_hbm, m_hbm, src_hbm, dst_hbm, a_out,
              srcv, dstv, elv, erv, mspv, av):
    c = lax.axis_index("c")
    s = lax.axis_index("s")

    pltpu.sync_copy(src_hbm.at[s], srcv)
    pltpu.sync_copy(dst_hbm.at[s], dstv)

    for j in range(2):
        h = 2 * c + j
        pltpu.sync_copy(el_hbm.at[h], elv)
        pltpu.sync_copy(er_hbm.at[h], erv)
        pltpu.sync_copy(m_hbm.at[h], mspv)
        msp = mspv[...]

        @pl.loop(0, C1)
        def _(jc):
            for g in range(6):
                sl = pl.ds(16 * g, 16)
                e = (plsc.load_gather(elv, [srcv[jc, sl]])
                     + plsc.load_gather(erv, [dstv[jc, sl]]))
                e = jnp.where(e > 0, e, NEG * e) - msp
                av[jc, sl] = jnp.exp(e)

        pltpu.sync_copy(av, a_out.at[h].at[s])


@functools.cache
def _sc0_kernel():
    return functools.partial(
        pl.kernel,
        out_type=jax.ShapeDtypeStruct((HEADS, 16, C1, CK), jnp.float32),
        mesh=plsc.VectorSubcoreMesh(core_axis_name="c", subcore_axis_name="s"),
        compiler_params=pltpu.CompilerParams(needs_layout_passes=False, use_tc_tiling_on_sc=False),
        scratch_types=[
            pltpu.VMEM((C1, CK), jnp.int32),
            pltpu.VMEM((C1, CK), jnp.int32),
            pltpu.VMEM((NP,), jnp.float32),
            pltpu.VMEM((NP,), jnp.float32),
            pltpu.VMEM((16,), jnp.float32),
            pltpu.VMEM((C1, CK), jnp.float32),
        ],
    )(_sc0_body)


def _sc0_call(*args):
    return _sc0_kernel()(*args)


def _sc1_body(z_hbm, a_hbm, src_hbm, dst_hbm,
              acc_out, den_out,
              srcv, dstv, av0, av1, idxo0, idxo1, rows0, rows1, zden,
              acc_sh, den_sh, gs0, gs1, ss0, ss1, ds0, ds1, as0, as1):
    c = lax.axis_index("c")
    s = lax.axis_index("s")
    rowsb = (rows0, rows1)
    idxob = (idxo0, idxo1)
    avb = (av0, av1)
    gsem = (gs0, gs1)
    ssem = (ss0, ss1)
    dsem = (ds0, ds1)
    asem = (as0, as1)

    pltpu.sync_copy(src_hbm.at[s], srcv)
    pltpu.sync_copy(dst_hbm.at[s], dstv)

    def zero_rows(r):
        @pl.loop(0, CK)
        def _(i):
            for g in range(HID // 16):
                r[i, pl.ds(16 * g, 16)] = jnp.zeros((16,), jnp.float32)

    zero_rows(rows0)
    zero_rows(rows1)

    @pl.loop(0, 40)
    def _(g):
        zden[pl.ds(16 * g, 16)] = jnp.zeros((16,), jnp.float32)

    for j in range(2):
        h = 2 * c + j
        hoff = h * NP

        @pl.loop(0, 6)
        def _(b):
            pltpu.sync_copy(rows0, acc_sh.at[pl.ds(s * 640 + b * CK, CK)])

        pltpu.sync_copy(rows0.at[pl.ds(0, 64)],
                        acc_sh.at[pl.ds(s * 640 + 6 * CK, 64)])
        pltpu.sync_copy(zden, den_sh.at[pl.ds(s * 640, 640)])
        plsc.subcore_barrier()

        ah = a_hbm.at[h].at[s]

        def gpart(k, b):
            pltpu.make_async_copy(rowsb[b], acc_sh.at[dstv.at[0]],
                                  ssem[b]).wait()
            pltpu.make_async_copy(avb[b], den_sh.at[dstv.at[0]],
                                  dsem[b]).wait()
            for g in range(CK // 16):
                sl = pl.ds(16 * g, 16)
                idxob[b][sl] = srcv[k, sl] + hoff
            pltpu.async_copy(z_hbm.at[idxob[b]], rowsb[b], gsem[b])
            pltpu.async_copy(ah.at[k], avb[b], asem[b])

        def spart(k, b):
            pltpu.make_async_copy(z_hbm.at[idxob[b]], rowsb[b],
                                  gsem[b]).wait()
            pltpu.make_async_copy(ah.at[k], avb[b], asem[b]).wait()
            r = rowsb[b]
            a = avb[b]

            @pl.loop(0, CK)
            def _(i):
                ai = plsc.load_gather(a, [jnp.full((16,), 0, jnp.int32) + i])
                for g in range(HID // 16):
                    sl = pl.ds(16 * g, 16)
                    r[i, sl] = r[i, sl] * ai

            pltpu.async_copy(r, acc_sh.at[dstv.at[k]], ssem[b], priority=1, add=True)
            pltpu.async_copy(a, den_sh.at[dstv.at[k]], dsem[b], priority=1, add=True)

        pltpu.async_copy(rows0, acc_sh.at[dstv.at[0]], ss0, add=True)
        pltpu.async_copy(rows1, acc_sh.at[dstv.at[0]], ss1, add=True)
        pltpu.async_copy(zden.at[pl.ds(0, CK)], den_sh.at[dstv.at[0]],
                         ds0, add=True)
        pltpu.async_copy(zden.at[pl.ds(0, CK)], den_sh.at[dstv.at[0]],
                         ds1, add=True)

        gpart(0, 0)
        gpart(1, 1)
        spart(0, 0)

        @pl.loop(0, (C1 - 2) // 2)
        def _(jj):
            k0 = 2 * jj + 2
            gpart(k0, 0)
            spart(k0 - 1, 1)
            gpart(k0 + 1, 1)
            spart(k0, 0)

        spart(C1 - 1, 1)
        for b in range(2):
            pltpu.make_async_copy(rowsb[b], acc_sh.at[dstv.at[0]],
                                  ssem[b]).wait()
            pltpu.make_async_copy(avb[b], den_sh.at[dstv.at[0]],
                                  dsem[b]).wait()
        plsc.subcore_barrier()

        @pl.loop(0, 6)
        def _(b):
            sl = pl.ds(s * 640 + b * CK, CK)
            pltpu.sync_copy(acc_sh.at[sl], acc_out.at[h].at[sl])

        sl64 = pl.ds(s * 640 + 6 * CK, 64)
        pltpu.sync_copy(acc_sh.at[sl64], acc_out.at[h].at[sl64])
        pltpu.sync_copy(den_sh.at[pl.ds(s * 640, 640)],
                        den_out.at[h].at[pl.ds(s * 640, 640)])
        plsc.subcore_barrier()

        if j == 0:
            zero_rows(rows0)
            zero_rows(rows1)


@functools.cache
def _sc1_kernel():
    return functools.partial(
        pl.kernel,
        out_type=(jax.ShapeDtypeStruct((HEADS, NP, HID), jnp.float32),
                  jax.ShapeDtypeStruct((HEADS, NP), jnp.float32)),
        mesh=plsc.VectorSubcoreMesh(core_axis_name="c", subcore_axis_name="s"),
        compiler_params=pltpu.CompilerParams(needs_layout_passes=False,
                                             use_tc_tiling_on_sc=False),
        scratch_types=[
            pltpu.VMEM((C1, CK), jnp.int32),
            pltpu.VMEM((C1, CK), jnp.int32),
            pltpu.VMEM((CK,), jnp.float32),
            pltpu.VMEM((CK,), jnp.float32),
            pltpu.VMEM((CK,), jnp.int32),
            pltpu.VMEM((CK,), jnp.int32),
            pltpu.VMEM((CK, HID), jnp.float32),
            pltpu.VMEM((CK, HID), jnp.float32),
            pltpu.VMEM((640,), jnp.float32),
            pltpu.VMEM_SHARED((NP, HID), jnp.float32),
            pltpu.VMEM_SHARED((NP,), jnp.float32),
            pltpu.SemaphoreType.DMA,
            pltpu.SemaphoreType.DMA,
            pltpu.SemaphoreType.DMA,
            pltpu.SemaphoreType.DMA,
            pltpu.SemaphoreType.DMA,
            pltpu.SemaphoreType.DMA,
            pltpu.SemaphoreType.DMA,
            pltpu.SemaphoreType.DMA,
        ],
    )(_sc1_body)


def _sc1_call(*args):
    return _sc1_kernel()(*args)


def _l2_body(thr_ref, acc_ref, den_ref, w_ref, s_ref, al_ref, ar_ref,
             z1_ref, el1_ref, er1_ref, m1l_ref, m1r_ref):
    parts = []
    for h in range(HEADS):
        d = den_ref[:, h:h + 1]
        x = acc_ref[h] / jnp.maximum(d, 1e-9)
        parts.append(jnp.where(x > 0, x, jnp.exp(jnp.minimum(x, 0.0)) - 1.0))
    h1 = jnp.concatenate(parts, axis=1)
    thr = thr_ref[0, 0]
    w = w_ref[...] * (s_ref[...] > thr).astype(jnp.float32)
    z1 = jnp.dot(h1, w, preferred_element_type=jnp.float32)
    el1 = jnp.sum(z1 * al_ref[...], axis=1, keepdims=True)
    er1 = jnp.sum(z1 * ar_ref[...], axis=1, keepdims=True)
    i = pl.program_id(0)
    ridx = i * 128 + lax.broadcasted_iota(jnp.int32, (128, 1), 0)
    valid = ridx < N
    el1 = jnp.where(valid, el1, -1e30)
    er1 = jnp.where(valid, er1, -1e30)
    z1_ref[...] = z1
    el1_ref[...] = el1
    er1_ref[...] = er1
    ml = jnp.max(el1)
    mr = jnp.max(er1)

    @pl.when(i == 0)
    def _():
        m1l_ref[0, 0] = ml
        m1r_ref[0, 0] = mr

    @pl.when(i > 0)
    def _():
        m1l_ref[0, 0] = jnp.maximum(m1l_ref[0, 0], ml)
        m1r_ref[0, 0] = jnp.maximum(m1r_ref[0, 0], mr)


def _l2_call(thr, acc0, den0T, W1, score1, attn_l1, attn_r1):
    grid = (NP // 128,)
    return pl.pallas_call(
        _l2_body,
        grid=grid,
        in_specs=[
            pl.BlockSpec(memory_space=pltpu.SMEM),
            pl.BlockSpec((HEADS, 128, HID), lambda i: (0, i, 0)),
            pl.BlockSpec((128, HEADS), lambda i: (i, 0)),
            pl.BlockSpec((HEADS * HID, NCLS), lambda i: (0, 0)),
            pl.BlockSpec((HEADS * HID, NCLS), lambda i: (0, 0)),
            pl.BlockSpec((1, NCLS), lambda i: (0, 0)),
            pl.BlockSpec((1, NCLS), lambda i: (0, 0)),
        ],
        out_specs=[
            pl.BlockSpec((128, NCLS), lambda i: (i, 0)),
            pl.BlockSpec((128, 1), lambda i: (i, 0)),
            pl.BlockSpec((128, 1), lambda i: (i, 0)),
            pl.BlockSpec(memory_space=pltpu.SMEM),
            pl.BlockSpec(memory_space=pltpu.SMEM),
        ],
        out_shape=[
            jax.ShapeDtypeStruct((NP, NCLS), jnp.float32),
            jax.ShapeDtypeStruct((NP, 1), jnp.float32),
            jax.ShapeDtypeStruct((NP, 1), jnp.float32),
            jax.ShapeDtypeStruct((1, 1), jnp.float32),
            jax.ShapeDtypeStruct((1, 1), jnp.float32),
        ],
    )(thr, acc0, den0T, W1, score1, attn_l1, attn_r1)


def _sc2_body(z_hbm, el_hbm, er_hbm, m_hbm, src_hbm, dst_hbm,
              acc_out, den_out,
              srcv, dstv, elv, erv, mspv, av0, av1, rows0, rows1, zden,
              acc_sh, den_sh, gs0, gs1, ss0, ss1, ds0, ds1):
    c = lax.axis_index("c")
    s = lax.axis_index("s")
    w = c * 16 + s
    rowsb = (rows0, rows1)
    avb = (av0, av1)
    gsem = (gs0, gs1)
    ssem = (ss0, ss1)
    dsem = (ds0, ds1)

    pltpu.sync_copy(src_hbm.at[w], srcv)
    pltpu.sync_copy(dst_hbm.at[w], dstv)

    def zero_rows(r):
        @pl.loop(0, CK)
        def _(i):
            for g in range(NCLS // 16):
                r[i, pl.ds(16 * g, 16)] = jnp.zeros((16,), jnp.float32)

    zero_rows(rows0)
    zero_rows(rows1)

    @pl.loop(0, 40)
    def _(g):
        zden[pl.ds(16 * g, 16)] = jnp.zeros((16,), jnp.float32)

    @pl.loop(0, 6)
    def _(b):
        pltpu.sync_copy(rows0, acc_sh.at[pl.ds(s * 640 + b * CK, CK)])

    pltpu.sync_copy(rows0.at[pl.ds(0, 64)],
                    acc_sh.at[pl.ds(s * 640 + 6 * CK, 64)])
    pltpu.sync_copy(zden, den_sh.at[pl.ds(s * 640, 640)])
    pltpu.sync_copy(el_hbm, elv)
    pltpu.sync_copy(er_hbm, erv)
    pltpu.sync_copy(m_hbm, mspv)
    plsc.subcore_barrier()
    msp = mspv[...]

    def gpart(k, b):
        pltpu.make_async_copy(rowsb[b], acc_sh.at[dstv.at[0]],
                              ssem[b]).wait()
        pltpu.make_async_copy(avb[b], den_sh.at[dstv.at[0]],
                              dsem[b]).wait()
        pltpu.async_copy(z_hbm.at[srcv.at[k]], rowsb[b], gsem[b])
        for g in range(CK // 16):
            sl = pl.ds(16 * g, 16)
            e = (plsc.load_gather(elv, [srcv[k, sl]])
                 + plsc.load_gather(erv, [dstv[k, sl]]))
            e = jnp.where(e > 0, e, NEG * e) - msp
            avb[b][sl] = jnp.exp(e)

    def spart(k, b):
        pltpu.make_async_copy(z_hbm.at[srcv.at[k]], rowsb[b],
                              gsem[b]).wait()
        r = rowsb[b]
        a = avb[b]

        @pl.loop(0, CK)
        def _(i):
            ai = plsc.load_gather(a, [jnp.full((16,), 0, jnp.int32) + i])
            for g in range(NCLS // 16):
                sl = pl.ds(16 * g, 16)
                r[i, sl] = r[i, sl] * ai

        pltpu.async_copy(r, acc_sh.at[dstv.at[k]], ssem[b], priority=1, add=True)
        pltpu.async_copy(a, den_sh.at[dstv.at[k]], dsem[b], priority=1, add=True)

    pltpu.async_copy(rows0, acc_sh.at[dstv.at[0]], ss0, add=True)
    pltpu.async_copy(rows1, acc_sh.at[dstv.at[0]], ss1, add=True)
    pltpu.async_copy(zden.at[pl.ds(0, CK)], den_sh.at[dstv.at[0]],
                     ds0, add=True)
    pltpu.async_copy(zden.at[pl.ds(0, CK)], den_sh.at[dstv.at[0]],
                     ds1, add=True)

    gpart(0, 0)
    gpart(1, 1)
    spart(0, 0)

    @pl.loop(0, (C2 - 2) // 2)
    def _(jj):
        k0 = 2 * jj + 2
        gpart(k0, 0)
        spart(k0 - 1, 1)
        gpart(k0 + 1, 1)
        spart(k0, 0)

    spart(C2 - 1, 1)
    for b in range(2):
        pltpu.make_async_copy(rowsb[b], acc_sh.at[dstv.at[0]],
                              ssem[b]).wait()
        pltpu.make_async_copy(avb[b], den_sh.at[dstv.at[0]],
                              dsem[b]).wait()
    plsc.subcore_barrier()

    @pl.loop(0, 6)
    def _(b):
        sl = pl.ds(s * 640 + b * CK, CK)
        pltpu.sync_copy(acc_sh.at[sl], acc_out.at[c].at[sl])

    sl64 = pl.ds(s * 640 + 6 * CK, 64)
    pltpu.sync_copy(acc_sh.at[sl64], acc_out.at[c].at[sl64])
    pltpu.sync_copy(den_sh.at[pl.ds(s * 640, 640)],
                    den_out.at[c].at[pl.ds(s * 640, 640)])


@functools.cache
def _sc2_kernel():
    return functools.partial(
        pl.kernel,
        out_type=(jax.ShapeDtypeStruct((2, NP, NCLS), jnp.float32),
                  jax.ShapeDtypeStruct((2, NP), jnp.float32)),
        mesh=plsc.VectorSubcoreMesh(core_axis_name="c", subcore_axis_name="s"),
        compiler_params=pltpu.CompilerParams(needs_layout_passes=False,
                                             use_tc_tiling_on_sc=False),
        scratch_types=[
            pltpu.VMEM((C2, CK), jnp.int32),
            pltpu.VMEM((C2, CK), jnp.int32),
            pltpu.VMEM((NP,), jnp.float32),
            pltpu.VMEM((NP,), jnp.float32),
            pltpu.VMEM((16,), jnp.float32),
            pltpu.VMEM((CK,), jnp.float32),
            pltpu.VMEM((CK,), jnp.float32),
            pltpu.VMEM((CK, NCLS), jnp.float32),
            pltpu.VMEM((CK, NCLS), jnp.float32),
            pltpu.VMEM((640,), jnp.float32),
            pltpu.VMEM_SHARED((NP, NCLS), jnp.float32),
            pltpu.VMEM_SHARED((NP,), jnp.float32),
            pltpu.SemaphoreType.DMA,
            pltpu.SemaphoreType.DMA,
            pltpu.SemaphoreType.DMA,
            pltpu.SemaphoreType.DMA,
            pltpu.SemaphoreType.DMA,
            pltpu.SemaphoreType.DMA,
        ],
    )(_sc2_body)


def _sc2_call(*args):
    return _sc2_kernel()(*args)


def _fin_body(acc_ref, den_ref, o_ref):
    num = acc_ref[0] + acc_ref[1]
    den = den_ref[:, 0:1] + den_ref[:, 1:2]
    o_ref[...] = num / jnp.maximum(den, 1e-9)


def _fin_call(acc1, den1T):
    grid = (NP // 128,)
    return pl.pallas_call(
        _fin_body,
        grid=grid,
        in_specs=[
            pl.BlockSpec((2, 128, NCLS), lambda i: (0, i, 0)),
            pl.BlockSpec((128, 2), lambda i: (i, 0)),
        ],
        out_specs=pl.BlockSpec((128, NCLS), lambda i: (i, 0)),
        out_shape=jax.ShapeDtypeStruct((NP, NCLS), jnp.float32),
    )(acc1, den1T)


def kernel(h, edge_index, W0, score0, attn_l0, attn_r0, W1, score1,
           attn_l1, attn_r1):
    f32 = jnp.float32
    i32 = jnp.int32

    scores = jnp.concatenate(
        [score0.reshape(-1), score1.reshape(-1)]).reshape(1280, 128)
    thr = _thr_call(scores)

    hp = jnp.pad(h, ((0, NP - N), (0, 0)))
    z0, el0, er0, elm, erm = _l1_call(thr, hp, W0, score0, attn_l0, attn_r0)
    msp0 = jnp.maximum(elm + erm, 0.0).reshape(HEADS, 1) * jnp.ones((1, 16), f32)

    loops = jnp.arange(N, dtype=i32)
    src = jnp.concatenate([edge_index[0], loops])
    dst = jnp.concatenate([edge_index[1], loops])
    pad1 = 16 * EP1 - ETOT
    padv = jnp.full((pad1,), N, i32)
    src1 = jnp.concatenate([src, padv]).reshape(16, C1, CK)
    dst1 = jnp.concatenate([dst, padv]).reshape(16, C1, CK)

    a0 = _sc0_call(el0.T, er0.T, msp0, src1, dst1)
    zflat = z0.reshape(HEADS * NP, HID)
    acc0, den0 = _sc1_call(zflat, a0, src1, dst1)

    z1, el1, er1, m1l, m1r = _l2_call(thr, acc0, den0.T, W1, score1,
                                      attn_l1, attn_r1)
    msp1 = jnp.maximum(m1l[0, 0] + m1r[0, 0], 0.0) * jnp.ones((16,), f32)

    src2 = jnp.concatenate([src, padv]).reshape(32, C2, CK)
    dst2 = jnp.concatenate([dst, padv]).reshape(32, C2, CK)
    acc1, den1 = _sc2_call(z1, el1.reshape(NP), er1.reshape(NP), msp1,
                           src2, dst2)

    out = _fin_call(acc1, den1.T)
    return out[:N]


# speedup vs baseline: 1.5004x; 1.0053x over previous
import functools

import jax
import jax.numpy as jnp
from jax import lax
from jax.experimental import pallas as pl
from jax.experimental.pallas import tpu as pltpu
from jax.experimental.pallas import tpu_sc as plsc

N = 10000
D_IN = 256
HID = 128
HEADS = 4
NCLS = 64
NEG = 0.2

NP = 10240
ETOT = 170000
CK = 96
C1 = 112
EP1 = C1 * CK
C2 = 56
EP2 = C2 * CK
KTH = 81921
I32MIN = jnp.iinfo(jnp.int32).min
I32MAX = jnp.iinfo(jnp.int32).max


def _l1_body(x_ref, w_ref, s_ref, s1_ref, al_ref, ar_ref,
             z_ref, el_ref, er_ref, elm_ref, erm_ref, thr_ref, thr_s):
    i = pl.program_id(0)

    @pl.when(i == 0)
    def _():
        b0 = lax.bitcast_convert_type(s_ref[...], jnp.int32)
        b1 = lax.bitcast_convert_type(s1_ref[...], jnp.int32)
        k0 = jnp.where(b0 >= 0, b0, I32MIN - b0)
        k1 = jnp.where(b1 >= 0, b1, I32MIN - b1)
        cnt_neg = (jnp.sum((k0 < 0).astype(jnp.int32))
                   + jnp.sum((k1 < 0).astype(jnp.int32)))
        in_neg = cnt_neg >= KTH
        lo0 = jnp.where(in_neg, I32MIN, 0).astype(jnp.int32)
        hi0 = jnp.where(in_neg, -1, I32MAX).astype(jnp.int32)

        def step(_, carry):
            lo, hi = carry
            mid = lo + lax.div(hi - lo, jnp.int32(2))
            cnt = (jnp.sum((k0 <= mid).astype(jnp.int32))
                   + jnp.sum((k1 <= mid).astype(jnp.int32)))
            ok = cnt >= KTH
            return (jnp.where(ok, lo, mid + 1), jnp.where(ok, mid, hi))

        lo, _hi = lax.fori_loop(0, 31, step, (lo0, hi0))
        tb = jnp.where(lo >= 0, lo, I32MIN - lo).reshape(1, 1)
        thr_val = lax.bitcast_convert_type(tb, jnp.float32)
        thr_s[0, 0] = thr_val[0, 0]
        thr_ref[0, 0] = thr_val[0, 0]

    thr = thr_s[0, 0]
    w = w_ref[...] * (s_ref[...] > thr).astype(jnp.float32)
    zb = jnp.dot(x_ref[...], w, preferred_element_type=jnp.float32)

    els, ers = [], []
    for h in range(HEADS):
        zh = zb[:, HID * h:HID * (h + 1)]
        els.append(jnp.sum(zh * al_ref[h:h + 1, :], axis=1, keepdims=True))
        ers.append(jnp.sum(zh * ar_ref[h:h + 1, :], axis=1, keepdims=True))
    el = jnp.concatenate(els, axis=1)
    er = jnp.concatenate(ers, axis=1)

    ridx = i * 128 + lax.broadcasted_iota(jnp.int32, (128, HEADS), 0)
    valid = ridx < N
    el = jnp.where(valid, el, -1e30)
    er = jnp.where(valid, er, -1e30)

    for h in range(HEADS):
        z_ref[h] = zb[:, HID * h:HID * (h + 1)]
    el_ref[...] = el
    er_ref[...] = er
    ml = jnp.max(el, axis=0, keepdims=True)
    mr = jnp.max(er, axis=0, keepdims=True)

    @pl.when(i == 0)
    def _():
        elm_ref[...] = ml
        erm_ref[...] = mr

    @pl.when(i > 0)
    def _():
        elm_ref[...] = jnp.maximum(elm_ref[...], ml)
        erm_ref[...] = jnp.maximum(erm_ref[...], mr)


def _l1_call(hp, W0, score0, score1, attn_l0, attn_r0):
    grid = (NP // 128,)
    return pl.pallas_call(
        _l1_body,
        grid=grid,
        in_specs=[
            pl.BlockSpec((128, D_IN), lambda i: (i, 0)),
            pl.BlockSpec((D_IN, HEADS * HID), lambda i: (0, 0)),
            pl.BlockSpec((D_IN, HEADS * HID), lambda i: (0, 0)),
            pl.BlockSpec((HEADS * HID, NCLS), lambda i: (0, 0)),
            pl.BlockSpec((HEADS, HID), lambda i: (0, 0)),
            pl.BlockSpec((HEADS, HID), lambda i: (0, 0)),
        ],
        out_specs=[
            pl.BlockSpec((HEADS, 128, HID), lambda i: (0, i, 0)),
            pl.BlockSpec((128, HEADS), lambda i: (i, 0)),
            pl.BlockSpec((128, HEADS), lambda i: (i, 0)),
            pl.BlockSpec((1, HEADS), lambda i: (0, 0)),
            pl.BlockSpec((1, HEADS), lambda i: (0, 0)),
            pl.BlockSpec(memory_space=pltpu.SMEM),
        ],
        out_shape=[
            jax.ShapeDtypeStruct((HEADS, NP, HID), jnp.float32),
            jax.ShapeDtypeStruct((NP, HEADS), jnp.float32),
            jax.ShapeDtypeStruct((NP, HEADS), jnp.float32),
            jax.ShapeDtypeStruct((1, HEADS), jnp.float32),
            jax.ShapeDtypeStruct((1, HEADS), jnp.float32),
            jax.ShapeDtypeStruct((1, 1), jnp.float32),
        ],
        scratch_shapes=[pltpu.SMEM((1, 1), jnp.float32)],
    )(hp, W0, score0, score1, attn_l0, attn_r0)


def _sc0_body(el_hbm, er_hbm, m_hbm, src_hbm, dst_hbm, a_out,
              srcv, dstv, elv, erv, mspv, av):
    c = lax.axis_index("c")
    s = lax.axis_index("s")

    pltpu.sync_copy(src_hbm.at[s], srcv)
    pltpu.sync_copy(dst_hbm.at[s], dstv)

    for j in range(2):
        h = 2 * c + j
        pltpu.sync_copy(el_hbm.at[h], elv)
        pltpu.sync_copy(er_hbm.at[h], erv)
        pltpu.sync_copy(m_hbm.at[h], mspv)
        msp = mspv[...]

        @pl.loop(0, C1)
        def _(jc):
            for g in range(6):
                sl = pl.ds(16 * g, 16)
                e = (plsc.load_gather(elv, [srcv[jc, sl]])
                     + plsc.load_gather(erv, [dstv[jc, sl]]))
                e = jnp.where(e > 0, e, NEG * e) - msp
                av[jc, sl] = jnp.exp(e)

        pltpu.sync_copy(av, a_out.at[h].at[s])


@functools.cache
def _sc0_kernel():
    return functools.partial(
        pl.kernel,
        out_type=jax.ShapeDtypeStruct((HEADS, 16, C1, CK), jnp.float32),
        mesh=plsc.VectorSubcoreMesh(core_axis_name="c", subcore_axis_name="s"),
        compiler_params=pltpu.CompilerParams(needs_layout_passes=False, use_tc_tiling_on_sc=False),
        scratch_types=[
            pltpu.VMEM((C1, CK), jnp.int32),
            pltpu.VMEM((C1, CK), jnp.int32),
            pltpu.VMEM((NP,), jnp.float32),
            pltpu.VMEM((NP,), jnp.float32),
            pltpu.VMEM((16,), jnp.float32),
            pltpu.VMEM((C1, CK), jnp.float32),
        ],
    )(_sc0_body)


def _sc0_call(*args):
    return _sc0_kernel()(*args)


def _sc1_body(z_hbm, a_hbm, src_hbm, dst_hbm,
              acc_out, den_out,
              srcv, dstv, av0, av1, idxo0, idxo1, rows0, rows1, zden,
              acc_sh, den_sh, gs0, gs1, ss0, ss1, ds0, ds1, as0, as1):
    c = lax.axis_index("c")
    s = lax.axis_index("s")
    rowsb = (rows0, rows1)
    idxob = (idxo0, idxo1)
    avb = (av0, av1)
    gsem = (gs0, gs1)
    ssem = (ss0, ss1)
    dsem = (ds0, ds1)
    asem = (as0, as1)

    pltpu.sync_copy(src_hbm.at[s], srcv)
    pltpu.sync_copy(dst_hbm.at[s], dstv)

    def zero_rows(r):
        @pl.loop(0, CK)
        def _(i):
            for g in range(HID // 16):
                r[i, pl.ds(16 * g, 16)] = jnp.zeros((16,), jnp.float32)

    zero_rows(rows0)
    zero_rows(rows1)

    @pl.loop(0, 40)
    def _(g):
        zden[pl.ds(16 * g, 16)] = jnp.zeros((16,), jnp.float32)

    for j in range(2):
        h = 2 * c + j
        hoff = h * NP

        @pl.loop(0, 6)
        def _(b):
            pltpu.sync_copy(rows0, acc_sh.at[pl.ds(s * 640 + b * CK, CK)])

        pltpu.sync_copy(rows0.at[pl.ds(0, 64)],
                        acc_sh.at[pl.ds(s * 640 + 6 * CK, 64)])
        pltpu.sync_copy(zden, den_sh.at[pl.ds(s * 640, 640)])
        plsc.subcore_barrier()

        ah = a_hbm.at[h].at[s]

        def gpart(k, b):
            pltpu.make_async_copy(rowsb[b], acc_sh.at[dstv.at[0]],
                                  ssem[b]).wait()
            pltpu.make_async_copy(avb[b], den_sh.at[dstv.at[0]],
                                  dsem[b]).wait()
            for g in range(CK // 16):
                sl = pl.ds(16 * g, 16)
                idxob[b][sl] = srcv[k, sl] + hoff
            pltpu.async_copy(z_hbm.at[idxob[b]], rowsb[b], gsem[b])
            pltpu.async_copy(ah.at[k], avb[b], asem[b])

        def spart(k, b):
            pltpu.make_async_copy(z_hbm.at[idxob[b]], rowsb[b],
                                  gsem[b]).wait()
            pltpu.make_async_copy(ah.at[k], avb[b], asem[b]).wait()
            r = rowsb[b]
            a = avb[b]

            @pl.loop(0, CK)
            def _(i):
                ai = plsc.load_gather(a, [jnp.full((16,), 0, jnp.int32) + i])
                for g in range(HID // 16):
                    sl = pl.ds(16 * g, 16)
                    r[i, sl] = r[i, sl] * ai

            pltpu.async_copy(r, acc_sh.at[dstv.at[k]], ssem[b], priority=1, add=True)
            pltpu.async_copy(a, den_sh.at[dstv.at[k]], dsem[b], priority=1, add=True)

        pltpu.async_copy(rows0, acc_sh.at[dstv.at[0]], ss0, add=True)
        pltpu.async_copy(rows1, acc_sh.at[dstv.at[0]], ss1, add=True)
        pltpu.async_copy(zden.at[pl.ds(0, CK)], den_sh.at[dstv.at[0]],
                         ds0, add=True)
        pltpu.async_copy(zden.at[pl.ds(0, CK)], den_sh.at[dstv.at[0]],
                         ds1, add=True)

        gpart(0, 0)
        gpart(1, 1)
        spart(0, 0)

        @pl.loop(0, (C1 - 2) // 2)
        def _(jj):
            k0 = 2 * jj + 2
            gpart(k0, 0)
            spart(k0 - 1, 1)
            gpart(k0 + 1, 1)
            spart(k0, 0)

        spart(C1 - 1, 1)
        for b in range(2):
            pltpu.make_async_copy(rowsb[b], acc_sh.at[dstv.at[0]],
                                  ssem[b]).wait()
            pltpu.make_async_copy(avb[b], den_sh.at[dstv.at[0]],
                                  dsem[b]).wait()
        plsc.subcore_barrier()

        @pl.loop(0, 6)
        def _(b):
            sl = pl.ds(s * 640 + b * CK, CK)
            pltpu.sync_copy(acc_sh.at[sl], acc_out.at[h].at[sl])

        sl64 = pl.ds(s * 640 + 6 * CK, 64)
        pltpu.sync_copy(acc_sh.at[sl64], acc_out.at[h].at[sl64])
        pltpu.sync_copy(den_sh.at[pl.ds(s * 640, 640)],
                        den_out.at[h].at[pl.ds(s * 640, 640)])
        plsc.subcore_barrier()

        if j == 0:
            zero_rows(rows0)
            zero_rows(rows1)


@functools.cache
def _sc1_kernel():
    return functools.partial(
        pl.kernel,
        out_type=(jax.ShapeDtypeStruct((HEADS, NP, HID), jnp.float32),
                  jax.ShapeDtypeStruct((HEADS, NP), jnp.float32)),
        mesh=plsc.VectorSubcoreMesh(core_axis_name="c", subcore_axis_name="s"),
        compiler_params=pltpu.CompilerParams(needs_layout_passes=False,
                                             use_tc_tiling_on_sc=False),
        scratch_types=[
            pltpu.VMEM((C1, CK), jnp.int32),
            pltpu.VMEM((C1, CK), jnp.int32),
            pltpu.VMEM((CK,), jnp.float32),
            pltpu.VMEM((CK,), jnp.float32),
            pltpu.VMEM((CK,), jnp.int32),
            pltpu.VMEM((CK,), jnp.int32),
            pltpu.VMEM((CK, HID), jnp.float32),
            pltpu.VMEM((CK, HID), jnp.float32),
            pltpu.VMEM((640,), jnp.float32),
            pltpu.VMEM_SHARED((NP, HID), jnp.float32),
            pltpu.VMEM_SHARED((NP,), jnp.float32),
            pltpu.SemaphoreType.DMA,
            pltpu.SemaphoreType.DMA,
            pltpu.SemaphoreType.DMA,
            pltpu.SemaphoreType.DMA,
            pltpu.SemaphoreType.DMA,
            pltpu.SemaphoreType.DMA,
            pltpu.SemaphoreType.DMA,
            pltpu.SemaphoreType.DMA,
        ],
    )(_sc1_body)


def _sc1_call(*args):
    return _sc1_kernel()(*args)


def _l2_body(thr_ref, acc_ref, den_ref, w_ref, s_ref, al_ref, ar_ref,
             z1_ref, el1_ref, er1_ref, m1l_ref, m1r_ref):
    parts = []
    for h in range(HEADS):
        d = den_ref[:, h:h + 1]
        x = acc_ref[h] / jnp.maximum(d, 1e-9)
        parts.append(jnp.where(x > 0, x, jnp.exp(jnp.minimum(x, 0.0)) - 1.0))
    h1 = jnp.concatenate(parts, axis=1)
    thr = thr_ref[0, 0]
    w = w_ref[...] * (s_ref[...] > thr).astype(jnp.float32)
    z1 = jnp.dot(h1, w, preferred_element_type=jnp.float32)
    el1 = jnp.sum(z1 * al_ref[...], axis=1, keepdims=True)
    er1 = jnp.sum(z1 * ar_ref[...], axis=1, keepdims=True)
    i = pl.program_id(0)
    ridx = i * 128 + lax.broadcasted_iota(jnp.int32, (128, 1), 0)
    valid = ridx < N
    el1 = jnp.where(valid, el1, -1e30)
    er1 = jnp.where(valid, er1, -1e30)
    z1_ref[...] = z1
    el1_ref[...] = el1
    er1_ref[...] = er1
    ml = jnp.max(el1)
    mr = jnp.max(er1)

    @pl.when(i == 0)
    def _():
        m1l_ref[0, 0] = ml
        m1r_ref[0, 0] = mr

    @pl.when(i > 0)
    def _():
        m1l_ref[0, 0] = jnp.maximum(m1l_ref[0, 0], ml)
        m1r_ref[0, 0] = jnp.maximum(m1r_ref[0, 0], mr)


def _l2_call(thr, acc0, den0T, W1, score1, attn_l1, attn_r1):
    grid = (NP // 128,)
    return pl.pallas_call(
        _l2_body,
        grid=grid,
        in_specs=[
            pl.BlockSpec(memory_space=pltpu.SMEM),
            pl.BlockSpec((HEADS, 128, HID), lambda i: (0, i, 0)),
            pl.BlockSpec((128, HEADS), lambda i: (i, 0)),
            pl.BlockSpec((HEADS * HID, NCLS), lambda i: (0, 0)),
            pl.BlockSpec((HEADS * HID, NCLS), lambda i: (0, 0)),
            pl.BlockSpec((1, NCLS), lambda i: (0, 0)),
            pl.BlockSpec((1, NCLS), lambda i: (0, 0)),
        ],
        out_specs=[
            pl.BlockSpec((128, NCLS), lambda i: (i, 0)),
            pl.BlockSpec((128, 1), lambda i: (i, 0)),
            pl.BlockSpec((128, 1), lambda i: (i, 0)),
            pl.BlockSpec(memory_space=pltpu.SMEM),
            pl.BlockSpec(memory_space=pltpu.SMEM),
        ],
        out_shape=[
            jax.ShapeDtypeStruct((NP, NCLS), jnp.float32),
            jax.ShapeDtypeStruct((NP, 1), jnp.float32),
            jax.ShapeDtypeStruct((NP, 1), jnp.float32),
            jax.ShapeDtypeStruct((1, 1), jnp.float32),
            jax.ShapeDtypeStruct((1, 1), jnp.float32),
        ],
    )(thr, acc0, den0T, W1, score1, attn_l1, attn_r1)


def _sc2_body(z_hbm, el_hbm, er_hbm, m_hbm, src_hbm, dst_hbm,
              acc_out, den_out,
              srcv, dstv, elv, erv, mspv, av0, av1, rows0, rows1, zden,
              acc_sh, den_sh, gs0, gs1, ss0, ss1, ds0, ds1):
    c = lax.axis_index("c")
    s = lax.axis_index("s")
    w = c * 16 + s
    rowsb = (rows0, rows1)
    avb = (av0, av1)
    gsem = (gs0, gs1)
    ssem = (ss0, ss1)
    dsem = (ds0, ds1)

    pltpu.sync_copy(src_hbm.at[w], srcv)
    pltpu.sync_copy(dst_hbm.at[w], dstv)

    def zero_rows(r):
        @pl.loop(0, CK)
        def _(i):
            for g in range(NCLS // 16):
                r[i, pl.ds(16 * g, 16)] = jnp.zeros((16,), jnp.float32)

    zero_rows(rows0)
    zero_rows(rows1)

    @pl.loop(0, 40)
    def _(g):
        zden[pl.ds(16 * g, 16)] = jnp.zeros((16,), jnp.float32)

    @pl.loop(0, 6)
    def _(b):
        pltpu.sync_copy(rows0, acc_sh.at[pl.ds(s * 640 + b * CK, CK)])

    pltpu.sync_copy(rows0.at[pl.ds(0, 64)],
                    acc_sh.at[pl.ds(s * 640 + 6 * CK, 64)])
    pltpu.sync_copy(zden, den_sh.at[pl.ds(s * 640, 640)])
    pltpu.sync_copy(el_hbm, elv)
    pltpu.sync_copy(er_hbm, erv)
    pltpu.sync_copy(m_hbm, mspv)
    plsc.subcore_barrier()
    msp = mspv[...]

    def gpart(k, b):
        pltpu.make_async_copy(rowsb[b], acc_sh.at[dstv.at[0]],
                              ssem[b]).wait()
        pltpu.make_async_copy(avb[b], den_sh.at[dstv.at[0]],
                              dsem[b]).wait()
        pltpu.async_copy(z_hbm.at[srcv.at[k]], rowsb[b], gsem[b])
        for g in range(CK // 16):
            sl = pl.ds(16 * g, 16)
            e = (plsc.load_gather(elv, [srcv[k, sl]])
                 + plsc.load_gather(erv, [dstv[k, sl]]))
            e = jnp.where(e > 0, e, NEG * e) - msp
            avb[b][sl] = jnp.exp(e)

    def spart(k, b):
        pltpu.make_async_copy(z_hbm.at[srcv.at[k]], rowsb[b],
                              gsem[b]).wait()
        r = rowsb[b]
        a = avb[b]

        @pl.loop(0, CK)
        def _(i):
            ai = plsc.load_gather(a, [jnp.full((16,), 0, jnp.int32) + i])
            for g in range(NCLS // 16):
                sl = pl.ds(16 * g, 16)
                r[i, sl] = r[i, sl] * ai

        pltpu.async_copy(r, acc_sh.at[dstv.at[k]], ssem[b], priority=1, add=True)
        pltpu.async_copy(a, den_sh.at[dstv.at[k]], dsem[b], priority=1, add=True)

    pltpu.async_copy(rows0, acc_sh.at[dstv.at[0]], ss0, add=True)
    pltpu.async_copy(rows1, acc_sh.at[dstv.at[0]], ss1, add=True)
    pltpu.async_copy(zden.at[pl.ds(0, CK)], den_sh.at[dstv.at[0]],
                     ds0, add=True)
    pltpu.async_copy(zden.at[pl.ds(0, CK)], den_sh.at[dstv.at[0]],
                     ds1, add=True)

    gpart(0, 0)
    gpart(1, 1)
    spart(0, 0)

    @pl.loop(0, (C2 - 2) // 2)
    def _(jj):
        k0 = 2 * jj + 2
        gpart(k0, 0)
        spart(k0 - 1, 1)
        gpart(k0 + 1, 1)
        spart(k0, 0)

    spart(C2 - 1, 1)
    for b in range(2):
        pltpu.make_async_copy(rowsb[b], acc_sh.at[dstv.at[0]],
                              ssem[b]).wait()
        pltpu.make_async_copy(avb[b], den_sh.at[dstv.at[0]],
                              dsem[b]).wait()
    plsc.subcore_barrier()

    @pl.loop(0, 6)
    def _(b):
        sl = pl.ds(s * 640 + b * CK, CK)
        pltpu.sync_copy(acc_sh.at[sl], acc_out.at[c].at[sl])

    sl64 = pl.ds(s * 640 + 6 * CK, 64)
    pltpu.sync_copy(acc_sh.at[sl64], acc_out.at[c].at[sl64])
    pltpu.sync_copy(den_sh.at[pl.ds(s * 640, 640)],
                    den_out.at[c].at[pl.ds(s * 640, 640)])


@functools.cache
def _sc2_kernel():
    return functools.partial(
        pl.kernel,
        out_type=(jax.ShapeDtypeStruct((2, NP, NCLS), jnp.float32),
                  jax.ShapeDtypeStruct((2, NP), jnp.float32)),
        mesh=plsc.VectorSubcoreMesh(core_axis_name="c", subcore_axis_name="s"),
        compiler_params=pltpu.CompilerParams(needs_layout_passes=False,
                                             use_tc_tiling_on_sc=False),
        scratch_types=[
            pltpu.VMEM((C2, CK), jnp.int32),
            pltpu.VMEM((C2, CK), jnp.int32),
            pltpu.VMEM((NP,), jnp.float32),
            pltpu.VMEM((NP,), jnp.float32),
            pltpu.VMEM((16,), jnp.float32),
            pltpu.VMEM((CK,), jnp.float32),
            pltpu.VMEM((CK,), jnp.float32),
            pltpu.VMEM((CK, NCLS), jnp.float32),
            pltpu.VMEM((CK, NCLS), jnp.float32),
            pltpu.VMEM((640,), jnp.float32),
            pltpu.VMEM_SHARED((NP, NCLS), jnp.float32),
            pltpu.VMEM_SHARED((NP,), jnp.float32),
            pltpu.SemaphoreType.DMA,
            pltpu.SemaphoreType.DMA,
            pltpu.SemaphoreType.DMA,
            pltpu.SemaphoreType.DMA,
            pltpu.SemaphoreType.DMA,
            pltpu.SemaphoreType.DMA,
        ],
    )(_sc2_body)


def _sc2_call(*args):
    return _sc2_kernel()(*args)


def _fin_body(acc_ref, den_ref, o_ref):
    num = acc_ref[0] + acc_ref[1]
    den = den_ref[:, 0:1] + den_ref[:, 1:2]
    o_ref[...] = num / jnp.maximum(den, 1e-9)


def _fin_call(acc1, den1T):
    grid = (NP // 128,)
    return pl.pallas_call(
        _fin_body,
        grid=grid,
        in_specs=[
            pl.BlockSpec((2, 128, NCLS), lambda i: (0, i, 0)),
            pl.BlockSpec((128, 2), lambda i: (i, 0)),
        ],
        out_specs=pl.BlockSpec((128, NCLS), lambda i: (i, 0)),
        out_shape=jax.ShapeDtypeStruct((NP, NCLS), jnp.float32),
    )(acc1, den1T)


def kernel(h, edge_index, W0, score0, attn_l0, attn_r0, W1, score1,
           attn_l1, attn_r1):
    f32 = jnp.float32
    i32 = jnp.int32

    hp = jnp.pad(h, ((0, NP - N), (0, 0)))
    z0, el0, er0, elm, erm, thr = _l1_call(hp, W0, score0, score1,
                                           attn_l0, attn_r0)
    msp0 = jnp.maximum(elm + erm, 0.0).reshape(HEADS, 1) * jnp.ones((1, 16), f32)

    loops = jnp.arange(N, dtype=i32)
    src = jnp.concatenate([edge_index[0], loops])
    dst = jnp.concatenate([edge_index[1], loops])
    pad1 = 16 * EP1 - ETOT
    padv = jnp.full((pad1,), N, i32)
    src1 = jnp.concatenate([src, padv]).reshape(16, C1, CK)
    dst1 = jnp.concatenate([dst, padv]).reshape(16, C1, CK)

    a0 = _sc0_call(el0.T, er0.T, msp0, src1, dst1)
    zflat = z0.reshape(HEADS * NP, HID)
    acc0, den0 = _sc1_call(zflat, a0, src1, dst1)

    z1, el1, er1, m1l, m1r = _l2_call(thr, acc0, den0.T, W1, score1,
                                      attn_l1, attn_r1)
    msp1 = jnp.maximum(m1l[0, 0] + m1r[0, 0], 0.0) * jnp.ones((16,), f32)

    src2 = jnp.concatenate([src, padv]).reshape(32, C2, CK)
    dst2 = jnp.concatenate([dst, padv]).reshape(32, C2, CK)
    acc1, den1 = _sc2_call(z1, el1.reshape(NP), er1.reshape(NP), msp1,
                           src2, dst2)

    out = _fin_call(acc1, den1.T)
    return out[:N]
